# baseline (speedup 1.0000x reference)
import functools

import jax
import jax.numpy as jnp
from jax.experimental import pallas as pl
from jax.experimental.pallas import tpu as pltpu

N = 10000
D = 128
BM = 200


def _sage_kernel(feat_ref, adja_ref, adjb_ref, feats_ref, w1_ref, w2_ref,
                 out_ref):
    feats = feats_ref[...]
    w1 = w1_ref[...]
    w2 = w2_ref[...]
    for j, adj_ref in enumerate((adja_ref, adjb_ref)):
        adj = adj_ref[...]
        rowsum = jnp.sum(adj, axis=1, keepdims=True)
        neigh = jnp.dot(adj, feats, preferred_element_type=jnp.float32)
        scale = 1.0 / (rowsum + 1.0)
        rows = pl.ds(j * BM, BM)
        out_ref[rows, :] = (
            jnp.dot(feat_ref[rows, :], w1, preferred_element_type=jnp.float32)
            + jnp.dot(neigh * scale, w2, preferred_element_type=jnp.float32)
        )


@functools.partial(jax.jit, static_argnames=())
def kernel(features, adj, W):
    w1 = W[:, :D].T
    w2 = W[:, D:].T
    grid = (N // (2 * BM),)
    return pl.pallas_call(
        _sage_kernel,
        grid=grid,
        in_specs=[
            pl.BlockSpec((2 * BM, D), lambda i: (i, 0)),
            pl.BlockSpec((BM, N), lambda i: (2 * i, 0)),
            pl.BlockSpec((BM, N), lambda i: (2 * i + 1, 0)),
            pl.BlockSpec((N, D), lambda i: (0, 0)),
            pl.BlockSpec((D, D), lambda i: (0, 0)),
            pl.BlockSpec((D, D), lambda i: (0, 0)),
        ],
        out_specs=pl.BlockSpec((2 * BM, D), lambda i: (i, 0)),
        out_shape=jax.ShapeDtypeStruct((N, D), jnp.float32),
        compiler_params=pltpu.CompilerParams(
            dimension_semantics=("parallel",),
        ),
    )(features, adj, adj, features, w1, w2)


# speedup vs baseline: 1.1014x; 1.1014x over previous
import functools

import jax
import jax.numpy as jnp
from jax.experimental import pallas as pl
from jax.experimental.pallas import tpu as pltpu

N = 10000
D = 128
BM = 400
HALF = 200
NSTEPS = N // BM


def _copy(adj_hbm, buf, sems, step, slot, h):
    return pltpu.make_async_copy(
        adj_hbm.at[pl.ds(step * BM + h * HALF, HALF), :],
        buf.at[slot, pl.ds(h * HALF, HALF), :],
        sems.at[slot, h],
    )


def _sage_kernel(feat_blk_ref, adj_hbm, feats_ref, w1_ref, w2_ref, out_ref,
                 buf, sems):
    i = pl.program_id(0)
    slot = jax.lax.rem(i, 2)
    nxt = jax.lax.rem(i + 1, 2)

    @pl.when(i == 0)
    def _():
        for h in range(2):
            _copy(adj_hbm, buf, sems, 0, 0, h).start()

    @pl.when(i + 1 < NSTEPS)
    def _():
        for h in range(2):
            _copy(adj_hbm, buf, sems, i + 1, nxt, h).start()

    for h in range(2):
        _copy(adj_hbm, buf, sems, i, slot, h).wait()

    adj = buf[slot]
    rowsum = jnp.sum(adj, axis=1, keepdims=True)
    neigh = jnp.dot(adj, feats_ref[...], preferred_element_type=jnp.float32)
    scale = 1.0 / (rowsum + 1.0)
    out_ref[...] = (
        jnp.dot(feat_blk_ref[...], w1_ref[...], preferred_element_type=jnp.float32)
        + jnp.dot(neigh * scale, w2_ref[...], preferred_element_type=jnp.float32)
    )


@functools.partial(jax.jit, static_argnames=())
def kernel(features, adj, W):
    w1 = W[:, :D].T
    w2 = W[:, D:].T
    return pl.pallas_call(
        _sage_kernel,
        grid=(NSTEPS,),
        in_specs=[
            pl.BlockSpec((BM, D), lambda i: (i, 0)),
            pl.BlockSpec(memory_space=pltpu.MemorySpace.HBM),
            pl.BlockSpec((N, D), lambda i: (0, 0)),
            pl.BlockSpec((D, D), lambda i: (0, 0)),
            pl.BlockSpec((D, D), lambda i: (0, 0)),
        ],
        out_specs=pl.BlockSpec((BM, D), lambda i: (i, 0)),
        out_shape=jax.ShapeDtypeStruct((N, D), jnp.float32),
        scratch_shapes=[
            pltpu.VMEM((2, BM, N), jnp.float32),
            pltpu.SemaphoreType.DMA((2, 2)),
        ],
        compiler_params=pltpu.CompilerParams(
            dimension_semantics=("arbitrary",),
        ),
    )(features, adj, features, w1, w2)
